# baseline (speedup 1.0000x reference)
import functools

import jax
import jax.numpy as jnp
import numpy as np
from jax import lax
from jax.experimental import pallas as pl
from jax.experimental.pallas import tpu as pltpu
from jax.experimental.pallas import tpu_sc as plsc

_S = 30
_C = 2 * _S + 1
_N = 1048576
_NW = 32
_COLS_W = _N // _NW
_R = 512
_CHUNKS = _COLS_W // _R
_MAGIC = np.int32(0x5F3759DF)


def _transform(xv):
    ax = jnp.abs(xv)
    y = ax + 1.0
    i = lax.bitcast_convert_type(y, jnp.int32)
    r = lax.bitcast_convert_type(
        _MAGIC - lax.shift_right_arithmetic(i, 1), jnp.float32
    )
    for _ in range(3):
        r = r * (1.5 - 0.5 * y * r * r)
    s = y * r
    h = jnp.sign(xv) * (s - 1.0) + 1e-3 * xv
    return jnp.clip(h, -float(_S), float(_S)) + float(_S)


def _body(x_hbm, out_hbm, xbuf, tile, sem):
    wid = lax.axis_index("s") * 2 + lax.axis_index("c")
    base = wid * _COLS_W
    pltpu.sync_copy(x_hbm.at[pl.ds(base, _COLS_W)], xbuf)

    zeros16 = jnp.zeros((16,), jnp.float32)
    lane = lax.iota(jnp.int32, 16)

    def chunk(c, carry):
        def zplane(p, cc):
            for j in range(_R // 16):
                tile[p, pl.ds(j * 16, 16)] = zeros16
            return cc

        lax.fori_loop(0, _C, zplane, 0)

        def group(g, cc):
            xv = xbuf[pl.ds(c * _R + g * 16, 16)]
            t = _transform(xv)
            fi = t.astype(jnp.int32)
            tf = t - fi.astype(jnp.float32)
            colv = g * 16 + lane
            plsc.store_scatter(tile, [fi, colv], 1.0 - tf)
            plsc.store_scatter(tile, [fi + 1, colv], tf, mask=fi < (_C - 1))
            return cc

        lax.fori_loop(0, _R // 16, group, 0)

        pltpu.sync_copy(tile, out_hbm.at[:, pl.ds(base + c * _R, _R)])
        return carry

    lax.fori_loop(0, _CHUNKS, chunk, 0)


def kernel(x):
    mesh = plsc.VectorSubcoreMesh(core_axis_name="c", subcore_axis_name="s")
    f = functools.partial(
        pl.kernel,
        mesh=mesh,
        compiler_params=pltpu.CompilerParams(needs_layout_passes=False),
        out_type=jax.ShapeDtypeStruct((_C, _N), jnp.float32),
        scratch_types=[
            pltpu.VMEM((_COLS_W,), jnp.float32),
            pltpu.VMEM((_C, _R), jnp.float32),
            pltpu.SemaphoreType.DMA,
        ],
    )(_body)
    return f(x).T


# speedup vs baseline: 2.8005x; 2.8005x over previous
import jax
import jax.numpy as jnp
from jax.experimental import pallas as pl

_S = 30
_EPS = 1e-3
_C = 2 * _S + 1
_BN = 32768
_BNL = _BN // 8


def _two_hot_body(x_ref, out_ref):
    x = x_ref[...]
    h = jnp.sign(x) * (jnp.sqrt(jnp.abs(x) + 1.0) - 1.0) + _EPS * x
    t = jnp.clip(h, -float(_S), float(_S)) + float(_S)
    t = t.reshape(1, _BN)
    col = jax.lax.broadcasted_iota(jnp.int32, (_C, 1), 0).astype(jnp.float32)
    out_ref[...] = jnp.maximum(1.0 - jnp.abs(t - col), 0.0)


def kernel(x):
    n = x.shape[0]
    g = n // _BN
    xg = x.reshape(g * 8, _BNL)
    out_t = pl.pallas_call(
        _two_hot_body,
        grid=(g,),
        in_specs=[pl.BlockSpec((8, _BNL), lambda j: (j, 0))],
        out_specs=pl.BlockSpec((_C, _BN), lambda j: (0, j)),
        out_shape=jax.ShapeDtypeStruct((_C, n), jnp.float32),
    )(xg)
    return out_t.T
